# baseline (speedup 1.0000x reference)
import dataclasses
import functools

import jax
import jax.numpy as jnp
from jax import lax
from jax.experimental import pallas as pl
from jax.experimental.pallas import tpu as pltpu
from jax.experimental.pallas import tpu_sc as plsc

DROP_FRAME_PROB = 0.125
NUM_SC_CORES = 2
NUM_SC_SUBCORES = 16
LANES = 16
NBUF = 6
CHUNKS = 8
DEPTH = 3


def _src_indices(T):
    rkey = jax.random.key(42)
    kdrop, kdir = jax.random.split(rkey)
    u_drop = jax.random.uniform(kdrop, (T,))
    u_dir = jax.random.uniform(kdir, (T,))
    drop = u_drop < DROP_FRAME_PROB
    diff = jnp.where(u_dir < 0.5, -1, 1)
    idx = jnp.arange(T)
    return jnp.where(drop, (idx + diff) % T, idx).astype(jnp.int32)


def kernel(frames, mask):
    T = frames.shape[0]
    row = 1
    for d in frames.shape[1:]:
        row *= d
    frames2 = frames.reshape(T, row)
    src = _src_indices(T)
    n_workers = NUM_SC_CORES * NUM_SC_SUBCORES
    per_w = T // n_workers
    W = row // CHUNKS

    mesh = plsc.VectorSubcoreMesh(core_axis_name="c", subcore_axis_name="s")

    cp = pltpu.CompilerParams()
    if "needs_layout_passes" in pltpu.CompilerParams.__dataclass_fields__:
        cp = dataclasses.replace(cp, needs_layout_passes=False)

    @functools.partial(
        pl.kernel,
        compiler_params=cp,
        out_type=jax.ShapeDtypeStruct((T, row), frames2.dtype),
        mesh=mesh,
        scratch_types=[
            pltpu.VMEM((T,), jnp.int32),
            pltpu.VMEM((NBUF * W,), frames2.dtype),
            pltpu.SemaphoreType.DMA,
            pltpu.SemaphoreType.DMA((NBUF,)),
            pltpu.SemaphoreType.DMA((NBUF,)),
        ],
    )
    def gather_rows(src_hbm, frames_hbm, out_hbm, idx_v, buf, sem_i, sem_in,
                    sem_out):
        wid = lax.axis_index("s") * NUM_SC_CORES + lax.axis_index("c")
        base = wid * per_w
        pltpu.async_copy(src_hbm, idx_v, sem_i).wait()

        iota = lax.iota(jnp.int32, LANES)
        neg = jnp.int32(-(2**31) + 1)
        vecs = [idx_v[pl.ds(j * LANES, LANES)] for j in range(T // LANES)]
        srcs = []
        for i in range(per_w):
            fi = base + i
            s = neg
            for j, vec in enumerate(vecs):
                m = (iota + (j * LANES)) == fi
                s = jnp.maximum(s, jnp.max(jnp.where(m, vec, neg)))
            srcs.append(s)

        copies = []
        for i in range(per_w):
            copies.append(
                pltpu.async_copy(
                    frames_hbm.at[srcs[i]],
                    out_hbm.at[base + i],
                    sem_out.at[i % NBUF],
                )
            )
        for c in copies:
            c.wait()

    out = gather_rows(src, frames2)
    return (out.reshape(frames.shape), mask)


# speedup vs baseline: 11.8167x; 11.8167x over previous
import dataclasses
import functools

import jax
import jax.numpy as jnp
from jax import lax
from jax.experimental import pallas as pl
from jax.experimental.pallas import tpu as pltpu
from jax.experimental.pallas import tpu_sc as plsc

DROP_FRAME_PROB = 0.125
NUM_SC_CORES = 2
NUM_SC_SUBCORES = 16
LANES = 16
NBUF = 6
CHUNKS = 8
DEPTH = 3


def _src_indices(T):
    rkey = jax.random.key(42)
    kdrop, kdir = jax.random.split(rkey)
    u_drop = jax.random.uniform(kdrop, (T,))
    u_dir = jax.random.uniform(kdir, (T,))
    drop = u_drop < DROP_FRAME_PROB
    diff = jnp.where(u_dir < 0.5, -1, 1)
    idx = jnp.arange(T)
    return jnp.where(drop, (idx + diff) % T, idx).astype(jnp.int32)


def kernel(frames, mask):
    T = frames.shape[0]
    row = 1
    for d in frames.shape[1:]:
        row *= d
    frames2 = frames.reshape(T, row)
    src = _src_indices(T)
    n_workers = NUM_SC_CORES * NUM_SC_SUBCORES
    per_w = T // n_workers
    W = row // CHUNKS

    mesh = plsc.VectorSubcoreMesh(core_axis_name="c", subcore_axis_name="s")

    cp = pltpu.CompilerParams()
    if "needs_layout_passes" in pltpu.CompilerParams.__dataclass_fields__:
        cp = dataclasses.replace(cp, needs_layout_passes=False)

    @functools.partial(
        pl.kernel,
        compiler_params=cp,
        out_type=jax.ShapeDtypeStruct((T, row), frames2.dtype),
        mesh=mesh,
        scratch_types=[
            pltpu.VMEM((T,), jnp.int32),
            pltpu.VMEM_SHARED((NUM_SC_SUBCORES * NBUF * W,), frames2.dtype),
            pltpu.SemaphoreType.DMA,
            pltpu.SemaphoreType.DMA((NBUF,)),
            pltpu.SemaphoreType.DMA((NBUF,)),
        ],
    )
    def gather_rows(src_hbm, frames_hbm, out_hbm, idx_v, buf, sem_i, sem_in,
                    sem_out):
        wid = lax.axis_index("s") * NUM_SC_CORES + lax.axis_index("c")
        base = wid * per_w
        pltpu.async_copy(src_hbm, idx_v, sem_i).wait()

        iota = lax.iota(jnp.int32, LANES)
        neg = jnp.int32(-(2**31) + 1)
        vecs = [idx_v[pl.ds(j * LANES, LANES)] for j in range(T // LANES)]
        srcs = []
        for i in range(per_w):
            fi = base + i
            s = neg
            for j, vec in enumerate(vecs):
                m = (iota + (j * LANES)) == fi
                s = jnp.maximum(s, jnp.max(jnp.where(m, vec, neg)))
            srcs.append(s)

        sid = lax.axis_index("s")
        buf_base = sid * (NBUF * W)
        items = [(i, c) for i in range(per_w) for c in range(CHUNKS)]
        n = len(items)
        in_h = [None] * n
        out_h = [None] * n

        def start_in(k):
            i, c = items[k]
            in_h[k] = pltpu.async_copy(
                frames_hbm.at[srcs[i], pl.ds(c * W, W)],
                buf.at[pl.ds(buf_base + (k % NBUF) * W, W)],
                sem_in.at[k % NBUF],
            )

        for k in range(min(DEPTH, n)):
            start_in(k)
        for k in range(n):
            i, c = items[k]
            in_h[k].wait()
            out_h[k] = pltpu.async_copy(
                buf.at[pl.ds(buf_base + (k % NBUF) * W, W)],
                out_hbm.at[base + i, pl.ds(c * W, W)],
                sem_out.at[k % NBUF],
            )
            j = k + DEPTH
            if j < n:
                if j >= NBUF:
                    out_h[j - NBUF].wait()
                start_in(j)
        for k in range(max(0, n - NBUF), n):
            out_h[k].wait()

    out = gather_rows(src, frames2)
    return (out.reshape(frames.shape), mask)
